# baseline (speedup 1.0000x reference)
import functools

import jax
import jax.numpy as jnp
import numpy as np
from jax import lax
from jax.experimental import pallas as pl
from jax.experimental.pallas import tpu as pltpu
from jax.experimental.pallas import tpu_sc as plsc

_VID = (3429, 3510, 3804, 3817, 3818, 1785, 2078, 3916, 4113,
        4314, 4261, 4321, 2364, 4513, 4702, 4740, 4801, 4808,
        3029, 1637, 4863, 5199, 5291, 5266, 5223, 2656, 2707,
        5382, 5615, 5710, 5658, 5635, 4136, 4079, 4152, 3976,
        4589, 4789, 4656, 4591, 5075, 5064, 5103, 5012, 5575,
        5700)

_B, _V, _C = 4096, 5711, 3
_A = len(_VID)
_G = 8
_NG = -(-_A // _G)
_NW = 32
_TASKS = [(c, g) for c in range(_C) for g in range(_NG)]

_VID_PAD = np.asarray(_VID + (_VID[-1],) * (_NG * _G - _A), dtype=np.int32)


def _sc_gather(vt, vid):
    mesh = plsc.VectorSubcoreMesh(core_axis_name="c", subcore_axis_name="s")
    nc = mesh.num_cores

    @functools.partial(
        pl.kernel,
        out_type=jax.ShapeDtypeStruct((_C, _A, _B), jnp.float32),
        mesh=mesh,
        scratch_types=[
            pltpu.VMEM((_NG * _G,), jnp.int32),
            pltpu.VMEM((_G, _B), jnp.float32),
            pltpu.SemaphoreType.DMA,
            pltpu.SemaphoreType.DMA,
        ],
    )
    def k(vt_hbm, vid_hbm, out_hbm, idx_v, buf_v, gsem, ssem):
        wid = lax.axis_index("s") * nc + lax.axis_index("c")
        pl.when(wid < len(_TASKS))(lambda: pltpu.sync_copy(vid_hbm, idx_v))
        half = _B // 2
        for t, (c, g) in enumerate(_TASKS):
            nr = min(_G, _A - g * _G)

            def issue(c=c, g=g, nr=nr):
                idx = idx_v.at[pl.ds(g * _G, _G)]
                g0 = pltpu.async_copy(
                    vt_hbm.at[c, :, pl.ds(0, half)].at[idx],
                    buf_v.at[:, pl.ds(0, half)], gsem)
                g1 = pltpu.async_copy(
                    vt_hbm.at[c, :, pl.ds(half, half)].at[idx],
                    buf_v.at[:, pl.ds(half, half)], gsem)
                g0.wait()
                s0 = pltpu.async_copy(
                    buf_v.at[pl.ds(0, nr), pl.ds(0, half)],
                    out_hbm.at[c, pl.ds(g * _G, nr), pl.ds(0, half)], ssem)
                g1.wait()
                s1 = pltpu.async_copy(
                    buf_v.at[pl.ds(0, nr), pl.ds(half, half)],
                    out_hbm.at[c, pl.ds(g * _G, nr), pl.ds(half, half)], ssem)
                s0.wait()
                s1.wait()

            pl.when(wid == t)(issue)

    return k(vt, vid)


def kernel(vertices):
    vt = jnp.transpose(vertices, (2, 1, 0))
    out_t = _sc_gather(vt, jnp.asarray(_VID_PAD))
    return jnp.transpose(out_t, (2, 1, 0))


# speedup vs baseline: 1.0241x; 1.0241x over previous
import functools

import jax
import jax.numpy as jnp
import numpy as np
from jax import lax
from jax.experimental import pallas as pl
from jax.experimental.pallas import tpu as pltpu
from jax.experimental.pallas import tpu_sc as plsc

_VID = (3429, 3510, 3804, 3817, 3818, 1785, 2078, 3916, 4113,
        4314, 4261, 4321, 2364, 4513, 4702, 4740, 4801, 4808,
        3029, 1637, 4863, 5199, 5291, 5266, 5223, 2656, 2707,
        5382, 5615, 5710, 5658, 5635, 4136, 4079, 4152, 3976,
        4589, 4789, 4656, 4591, 5075, 5064, 5103, 5012, 5575,
        5700)

_B, _V, _C = 4096, 5711, 3
_A = len(_VID)
_G = 8
_NG = -(-_A // _G)
_NW = 32
_TASKS = [(c, g) for c in range(_C) for g in range(_NG)]

_VID_PAD = np.asarray(_VID + (_VID[-1],) * (_NG * _G - _A), dtype=np.int32)


def _sc_gather(vt, vid):
    mesh = plsc.VectorSubcoreMesh(core_axis_name="c", subcore_axis_name="s")
    nc = mesh.num_cores

    @functools.partial(
        pl.kernel,
        out_type=jax.ShapeDtypeStruct((_C, _A, _B), jnp.float32),
        mesh=mesh,
        scratch_types=[
            pltpu.VMEM((_NG * _G,), jnp.int32),
            pltpu.VMEM((_G, _B), jnp.float32),
            pltpu.SemaphoreType.DMA,
            pltpu.SemaphoreType.DMA,
        ],
    )
    def k(vt_hbm, vid_hbm, out_hbm, idx_v, buf_v, gsem, ssem):
        wid = lax.axis_index("s") * nc + lax.axis_index("c")
        pl.when(wid < len(_TASKS))(lambda: pltpu.sync_copy(vid_hbm, idx_v))
        for t, (c, g) in enumerate(_TASKS):
            nr = min(_G, _A - g * _G)

            def issue(c=c, g=g, nr=nr):
                pltpu.async_copy(
                    vt_hbm.at[c].at[idx_v.at[pl.ds(g * _G, _G)]],
                    buf_v,
                    gsem,
                ).wait()
                pltpu.async_copy(
                    buf_v.at[pl.ds(0, nr), :],
                    out_hbm.at[c, pl.ds(g * _G, nr), :],
                    ssem,
                ).wait()

            pl.when(wid == t)(issue)

    return k(vt, vid)


def kernel(vertices):
    vt = jnp.transpose(vertices, (2, 1, 0))
    out_t = _sc_gather(vt, jnp.asarray(_VID_PAD))
    return jnp.transpose(out_t, (2, 1, 0))
